# baseline (speedup 1.0000x reference)
import jax
import jax.numpy as jnp
from jax import lax
from jax.experimental import pallas as pl
from jax.experimental.pallas import tpu as pltpu
from jax.experimental.pallas import tpu_sc as plsc

MIN_VAR_, MIN_MEAN_ = 5.5, 4.5
DELTA_VAR_, DELTA_MEAN_ = 0.95, 1.05
EPS_ = 1e-8
LN2_ = 0.6931471805599453
MAGIC_ = 0x3F3504F3
C1_, C2_, C3_, C4_, C5_, C6_ = (1.000013, -0.49978617, 0.33228943,
                                -0.25564772, 0.22294995, -0.13931262)
L_ = 16


def _vlog(x):
    bits = lax.bitcast_convert_type(x, jnp.int32)
    e = (bits - MAGIC_) >> 23
    m = lax.bitcast_convert_type(bits - (e << 23), jnp.float32)
    t = m - 1.0
    p = t * (C1_ + t * (C2_ + t * (C3_ + t * (C4_ + t * (C5_ + t * C6_)))))
    return e.astype(jnp.float32) * LN2_ + p


def _dg(v, idx):
    return v.at[idx].get(mode="promise_in_bounds")


def _make_sc_call(B):
    info = plsc.get_sparse_core_info()
    NC, NS = info.num_cores, info.num_subcores
    NW = NC * NS
    assert B % (NW * L_) == 0
    BPW = B // NW

    mesh = plsc.VectorSubcoreMesh(core_axis_name="c", subcore_axis_name="s")

    def body(buf_h, pos_h, neg_h, buf_v, tab_v, pos_v, neg_v, sem):
        wid = lax.axis_index("s") * NC + lax.axis_index("c")
        base = wid * BPW

        copies = [
            pltpu.async_copy(buf_h.at[pl.ds(wid * (5 * BPW), 5 * BPW)],
                             buf_v, sem),
            pltpu.async_copy(buf_h.at[pl.ds(5 * B, 2 * L_)], tab_v, sem),
        ]
        for c in copies:
            c.wait()

        tmin_s = tab_v[pl.ds(0, L_)] * MIN_VAR_ + MIN_MEAN_
        tmax_s = tmin_s + (jnp.abs(tab_v[pl.ds(L_, L_)]) * DELTA_VAR_
                           + DELTA_MEAN_)

        @plsc.parallel_loop(0, BPW, step=L_, unroll=4)
        def _step(i):
            sl = pl.ds(i, L_)
            x0 = buf_v[pl.ds(i, L_)]
            x1 = buf_v[pl.ds(BPW + i, L_)]
            ee0 = buf_v[pl.ds(2 * BPW + i, L_)]
            ee1 = buf_v[pl.ds(3 * BPW + i, L_)]

            idxv = lax.bitcast_convert_type(
                buf_v[pl.ds(4 * BPW + i, L_)], jnp.int32)
            i0 = idxv + idxv
            i1 = i0 + 1
            tm0 = _dg(tmin_s, i0)
            tm1 = _dg(tmin_s, i1)
            tx0 = _dg(tmax_s, i0)
            tx1 = _dg(tmax_s, i1)

            t1m0 = jnp.abs(x0) * MIN_VAR_ + MIN_MEAN_
            t1m1 = jnp.abs(x1) * MIN_VAR_ + MIN_MEAN_
            t1d0 = jnp.abs(ee0) * MIN_VAR_ + MIN_MEAN_
            t1d1 = jnp.abs(ee1) * MIN_VAR_ + MIN_MEAN_
            t1x0 = t1m0 + t1d0
            t1x1 = t1m1 + t1d1

            md0 = jnp.minimum(t1x0, tx0) - jnp.maximum(t1m0, tm0)
            md1 = jnp.minimum(t1x1, tx1) - jnp.maximum(t1m1, tm1)
            disjoint = (md0 <= 0.0) | (md1 <= 0.0)
            meetprod = jnp.maximum(md0, EPS_) * jnp.maximum(md1, EPS_)
            d = t1d0 * t1d1
            j = (jnp.maximum(t1x0, tx0) - jnp.minimum(t1m0, tm0)) * \
                (jnp.maximum(t1x1, tx1) - jnp.minimum(t1m1, tm1))
            t2p = (tx0 - tm0) * (tx1 - tm1)

            ld = _vlog(d)
            pos_arg = jnp.where(disjoint,
                                jnp.maximum(j - d - t2p, EPS_ * j), meetprod)
            neg_arg = jnp.where(disjoint, d,
                                jnp.maximum(d - meetprod, EPS_ * d))
            pos_v[sl] = ld - _vlog(pos_arg)
            neg_v[sl] = ld - _vlog(neg_arg)

        pltpu.sync_copy(pos_v, pos_h.at[pl.ds(base, BPW)])
        pltpu.sync_copy(neg_v, neg_h.at[pl.ds(base, BPW)])

    return pl.kernel(
        body,
        out_type=(jax.ShapeDtypeStruct((B,), jnp.float32),
                  jax.ShapeDtypeStruct((B,), jnp.float32)),
        mesh=mesh,
        scratch_types=[
            pltpu.VMEM((5 * BPW,), jnp.float32),
            pltpu.VMEM((2 * L_,), jnp.float32),
            pltpu.VMEM((BPW,), jnp.float32),
            pltpu.VMEM((BPW,), jnp.float32),
            pltpu.SemaphoreType.DMA,
        ],
    )


def kernel(t1x, t2_embed1, t2_embed2, min_feature_embed, delta_feature_embed):
    B = t1x.shape[0]
    info = plsc.get_sparse_core_info()
    NW = info.num_cores * info.num_subcores
    BPW = B // NW
    call = _make_sc_call(B)
    t1b = t1x.reshape(NW, BPW, 2).swapaxes(1, 2)
    eb = t2_embed1.reshape(NW, BPW, 2).swapaxes(1, 2)
    idxb = lax.bitcast_convert_type(
        t2_embed2.astype(jnp.int32), jnp.float32).reshape(NW, 1, BPW)
    blocks = jnp.concatenate([t1b, eb, idxb], axis=1).reshape(5 * B)
    buf = jnp.concatenate([blocks, min_feature_embed.reshape(L_),
                           delta_feature_embed.reshape(L_)])
    return call(buf)


# speedup vs baseline: 1.0032x; 1.0032x over previous
import jax
import jax.numpy as jnp
from jax import lax
from jax.experimental import pallas as pl
from jax.experimental.pallas import tpu as pltpu
from jax.experimental.pallas import tpu_sc as plsc

MIN_VAR_, MIN_MEAN_ = 5.5, 4.5
DELTA_VAR_, DELTA_MEAN_ = 0.95, 1.05
EPS_ = 1e-8
LN2_ = 0.6931471805599453
MAGIC_ = 0x3F3504F3
C1_, C2_, C3_, C4_, C5_, C6_ = (1.000013, -0.49978617, 0.33228943,
                                -0.25564772, 0.22294995, -0.13931262)
L_ = 16


def _vlog(x):
    bits = lax.bitcast_convert_type(x, jnp.int32)
    e = (bits - MAGIC_) >> 23
    m = lax.bitcast_convert_type(bits - (e << 23), jnp.float32)
    t = m - 1.0
    p = t * (C1_ + t * (C2_ + t * (C3_ + t * (C4_ + t * (C5_ + t * C6_)))))
    return e.astype(jnp.float32) * LN2_ + p


def _dg(v, idx):
    return v.at[idx].get(mode="promise_in_bounds")


def _make_sc_call(B):
    info = plsc.get_sparse_core_info()
    NC, NS = info.num_cores, info.num_subcores
    NW = NC * NS
    assert B % (NW * L_) == 0
    BPW = B // NW

    mesh = plsc.VectorSubcoreMesh(core_axis_name="c", subcore_axis_name="s")

    def body(buf_h, pos_h, neg_h, buf_v, tab_v, pos_v, neg_v, sem):
        wid = lax.axis_index("s") * NC + lax.axis_index("c")
        base = wid * BPW

        copies = [
            pltpu.async_copy(buf_h.at[pl.ds(wid * (5 * BPW), 5 * BPW)],
                             buf_v, sem),
            pltpu.async_copy(buf_h.at[pl.ds(5 * B, 2 * L_)], tab_v, sem),
        ]
        for c in copies:
            c.wait()

        tmin_s = tab_v[pl.ds(0, L_)] * MIN_VAR_ + MIN_MEAN_
        tmax_s = tmin_s + (jnp.abs(tab_v[pl.ds(L_, L_)]) * DELTA_VAR_
                           + DELTA_MEAN_)

        @plsc.parallel_loop(0, BPW, step=L_, unroll=4)
        def _step(i):
            sl = pl.ds(i, L_)
            x0 = buf_v[pl.ds(i, L_)]
            x1 = buf_v[pl.ds(BPW + i, L_)]
            ee0 = buf_v[pl.ds(2 * BPW + i, L_)]
            ee1 = buf_v[pl.ds(3 * BPW + i, L_)]

            idxv = buf_v[pl.ds(4 * BPW + i, L_)].astype(jnp.int32)
            i0 = idxv + idxv
            i1 = i0 + 1
            tm0 = _dg(tmin_s, i0)
            tm1 = _dg(tmin_s, i1)
            tx0 = _dg(tmax_s, i0)
            tx1 = _dg(tmax_s, i1)

            t1m0 = jnp.abs(x0) * MIN_VAR_ + MIN_MEAN_
            t1m1 = jnp.abs(x1) * MIN_VAR_ + MIN_MEAN_
            t1d0 = jnp.abs(ee0) * MIN_VAR_ + MIN_MEAN_
            t1d1 = jnp.abs(ee1) * MIN_VAR_ + MIN_MEAN_
            t1x0 = t1m0 + t1d0
            t1x1 = t1m1 + t1d1

            md0 = jnp.minimum(t1x0, tx0) - jnp.maximum(t1m0, tm0)
            md1 = jnp.minimum(t1x1, tx1) - jnp.maximum(t1m1, tm1)
            disjoint = (md0 <= 0.0) | (md1 <= 0.0)
            meetprod = jnp.maximum(md0, EPS_) * jnp.maximum(md1, EPS_)
            d = t1d0 * t1d1
            j = (jnp.maximum(t1x0, tx0) - jnp.minimum(t1m0, tm0)) * \
                (jnp.maximum(t1x1, tx1) - jnp.minimum(t1m1, tm1))
            t2p = (tx0 - tm0) * (tx1 - tm1)

            ld = _vlog(d)
            pos_arg = jnp.where(disjoint,
                                jnp.maximum(j - d - t2p, EPS_ * j), meetprod)
            neg_arg = jnp.where(disjoint, d,
                                jnp.maximum(d - meetprod, EPS_ * d))
            pos_v[sl] = ld - _vlog(pos_arg)
            neg_v[sl] = ld - _vlog(neg_arg)

        pltpu.sync_copy(pos_v, pos_h.at[pl.ds(base, BPW)])
        pltpu.sync_copy(neg_v, neg_h.at[pl.ds(base, BPW)])

    return pl.kernel(
        body,
        out_type=(jax.ShapeDtypeStruct((B,), jnp.float32),
                  jax.ShapeDtypeStruct((B,), jnp.float32)),
        mesh=mesh,
        scratch_types=[
            pltpu.VMEM((5 * BPW,), jnp.float32),
            pltpu.VMEM((2 * L_,), jnp.float32),
            pltpu.VMEM((BPW,), jnp.float32),
            pltpu.VMEM((BPW,), jnp.float32),
            pltpu.SemaphoreType.DMA,
        ],
    )


def kernel(t1x, t2_embed1, t2_embed2, min_feature_embed, delta_feature_embed):
    B = t1x.shape[0]
    info = plsc.get_sparse_core_info()
    NW = info.num_cores * info.num_subcores
    BPW = B // NW
    call = _make_sc_call(B)
    t1b = t1x.reshape(NW, BPW, 2).swapaxes(1, 2)
    eb = t2_embed1.reshape(NW, BPW, 2).swapaxes(1, 2)
    idxb = t2_embed2.astype(jnp.float32).reshape(NW, 1, BPW)
    blocks = jnp.concatenate([t1b, eb, idxb], axis=1).reshape(5 * B)
    buf = jnp.concatenate([blocks, min_feature_embed.reshape(L_),
                           delta_feature_embed.reshape(L_)])
    return call(buf)


# speedup vs baseline: 1.0518x; 1.0484x over previous
import jax
import jax.numpy as jnp
from jax import lax
from jax.experimental import pallas as pl
from jax.experimental.pallas import tpu as pltpu
from jax.experimental.pallas import tpu_sc as plsc

MIN_VAR_, MIN_MEAN_ = 5.5, 4.5
DELTA_VAR_, DELTA_MEAN_ = 0.95, 1.05
EPS_ = 1e-8
LN2_ = 0.6931471805599453
MAGIC_ = 0x3F3504F3
C1_, C2_, C3_, C4_, C5_, C6_ = (1.000013, -0.49978617, 0.33228943,
                                -0.25564772, 0.22294995, -0.13931262)
L_ = 16


def _vlog(x):
    bits = lax.bitcast_convert_type(x, jnp.int32)
    e = (bits - MAGIC_) >> 23
    m = lax.bitcast_convert_type(bits - (e << 23), jnp.float32)
    t = m - 1.0
    p = t * (C1_ + t * (C2_ + t * (C3_ + t * (C4_ + t * (C5_ + t * C6_)))))
    return e.astype(jnp.float32) * LN2_ + p


def _dg(v, idx):
    return v.at[idx].get(mode="promise_in_bounds")


def _make_sc_call(B):
    info = plsc.get_sparse_core_info()
    NC, NS = info.num_cores, info.num_subcores
    NW = NC * NS
    assert B % (NW * L_) == 0
    BPW = B // NW

    mesh = plsc.VectorSubcoreMesh(core_axis_name="c", subcore_axis_name="s")

    def body(t1_h, e_h, idx_h, tab_h, pos_h, neg_h,
             x0_v, x1_v, e0_v, e1_v, idx_v, tab_v, pos_v, neg_v, sem):
        wid = lax.axis_index("s") * NC + lax.axis_index("c")
        base = wid * BPW

        copies = [
            pltpu.async_copy(t1_h.at[pl.ds(base, BPW)], x0_v, sem),
            pltpu.async_copy(t1_h.at[pl.ds(B + base, BPW)], x1_v, sem),
            pltpu.async_copy(e_h.at[pl.ds(base, BPW)], e0_v, sem),
            pltpu.async_copy(e_h.at[pl.ds(B + base, BPW)], e1_v, sem),
            pltpu.async_copy(idx_h.at[pl.ds(base, BPW)], idx_v, sem),
            pltpu.async_copy(tab_h, tab_v, sem),
        ]
        for c in copies:
            c.wait()

        tmin_s = tab_v[pl.ds(0, L_)] * MIN_VAR_ + MIN_MEAN_
        tmax_s = tmin_s + (jnp.abs(tab_v[pl.ds(L_, L_)]) * DELTA_VAR_
                           + DELTA_MEAN_)

        @plsc.parallel_loop(0, BPW, step=L_, unroll=8)
        def _step(i):
            sl = pl.ds(i, L_)
            x0 = x0_v[sl]
            x1 = x1_v[sl]
            ee0 = e0_v[sl]
            ee1 = e1_v[sl]

            idxv = idx_v[sl]
            i0 = idxv + idxv
            i1 = i0 + 1
            tm0 = _dg(tmin_s, i0)
            tm1 = _dg(tmin_s, i1)
            tx0 = _dg(tmax_s, i0)
            tx1 = _dg(tmax_s, i1)

            t1m0 = jnp.abs(x0) * MIN_VAR_ + MIN_MEAN_
            t1m1 = jnp.abs(x1) * MIN_VAR_ + MIN_MEAN_
            t1d0 = jnp.abs(ee0) * MIN_VAR_ + MIN_MEAN_
            t1d1 = jnp.abs(ee1) * MIN_VAR_ + MIN_MEAN_
            t1x0 = t1m0 + t1d0
            t1x1 = t1m1 + t1d1

            md0 = jnp.minimum(t1x0, tx0) - jnp.maximum(t1m0, tm0)
            md1 = jnp.minimum(t1x1, tx1) - jnp.maximum(t1m1, tm1)
            disjoint = (md0 <= 0.0) | (md1 <= 0.0)
            meetprod = jnp.maximum(md0, EPS_) * jnp.maximum(md1, EPS_)
            d = t1d0 * t1d1
            j = (jnp.maximum(t1x0, tx0) - jnp.minimum(t1m0, tm0)) * \
                (jnp.maximum(t1x1, tx1) - jnp.minimum(t1m1, tm1))
            t2p = (tx0 - tm0) * (tx1 - tm1)

            ld = _vlog(d)
            pos_arg = jnp.where(disjoint,
                                jnp.maximum(j - d - t2p, EPS_ * j), meetprod)
            neg_arg = jnp.where(disjoint, d,
                                jnp.maximum(d - meetprod, EPS_ * d))
            pos_v[sl] = ld - _vlog(pos_arg)
            neg_v[sl] = ld - _vlog(neg_arg)

        outs = [
            pltpu.async_copy(pos_v, pos_h.at[pl.ds(base, BPW)], sem),
            pltpu.async_copy(neg_v, neg_h.at[pl.ds(base, BPW)], sem),
        ]
        for c in outs:
            c.wait()

    return pl.kernel(
        body,
        out_type=(jax.ShapeDtypeStruct((B,), jnp.float32),
                  jax.ShapeDtypeStruct((B,), jnp.float32)),
        mesh=mesh,
        scratch_types=[
            pltpu.VMEM((BPW,), jnp.float32),
            pltpu.VMEM((BPW,), jnp.float32),
            pltpu.VMEM((BPW,), jnp.float32),
            pltpu.VMEM((BPW,), jnp.float32),
            pltpu.VMEM((BPW,), jnp.int32),
            pltpu.VMEM((2 * L_,), jnp.float32),
            pltpu.VMEM((BPW,), jnp.float32),
            pltpu.VMEM((BPW,), jnp.float32),
            pltpu.SemaphoreType.DMA,
        ],
    )


def kernel(t1x, t2_embed1, t2_embed2, min_feature_embed, delta_feature_embed):
    B = t1x.shape[0]
    call = _make_sc_call(B)
    tab = jnp.concatenate([min_feature_embed.reshape(L_),
                           delta_feature_embed.reshape(L_)])
    return call(
        t1x.T.reshape(B * 2),
        t2_embed1.T.reshape(B * 2),
        t2_embed2.astype(jnp.int32),
        tab,
    )


# speedup vs baseline: 1.0643x; 1.0119x over previous
import jax
import jax.numpy as jnp
from jax import lax
from jax.experimental import pallas as pl
from jax.experimental.pallas import tpu as pltpu
from jax.experimental.pallas import tpu_sc as plsc

MIN_VAR_, MIN_MEAN_ = 5.5, 4.5
DELTA_VAR_, DELTA_MEAN_ = 0.95, 1.05
EPS_ = 1e-8
LN2_ = 0.6931471805599453
MAGIC_ = 0x3F3504F3
C1_, C2_, C3_, C4_, C5_, C6_ = (1.000013, -0.49978617, 0.33228943,
                                -0.25564772, 0.22294995, -0.13931262)
L_ = 16


def _vlog(x):
    bits = lax.bitcast_convert_type(x, jnp.int32)
    e = (bits - MAGIC_) >> 23
    m = lax.bitcast_convert_type(bits - (e << 23), jnp.float32)
    t = m - 1.0
    p = t * (C1_ + t * (C2_ + t * (C3_ + t * (C4_ + t * (C5_ + t * C6_)))))
    return e.astype(jnp.float32) * LN2_ + p


def _dg(v, idx):
    return v.at[idx].get(mode="promise_in_bounds")


def _make_sc_call(B):
    info = plsc.get_sparse_core_info()
    NC, NS = info.num_cores, info.num_subcores
    NW = NC * NS
    assert B % (NW * L_) == 0
    BPW = B // NW

    mesh = plsc.VectorSubcoreMesh(core_axis_name="c", subcore_axis_name="s")

    def body(t1_h, e_h, idx_h, tab_h, pos_h, neg_h,
             x0_v, x1_v, e0_v, e1_v, idx_v, tab_v, pos_v, neg_v, sem):
        wid = lax.axis_index("s") * NC + lax.axis_index("c")
        base = wid * BPW

        copies = [
            pltpu.async_copy(t1_h.at[pl.ds(base, BPW)], x0_v, sem),
            pltpu.async_copy(t1_h.at[pl.ds(B + base, BPW)], x1_v, sem),
            pltpu.async_copy(e_h.at[pl.ds(base, BPW)], e0_v, sem),
            pltpu.async_copy(e_h.at[pl.ds(B + base, BPW)], e1_v, sem),
            pltpu.async_copy(idx_h.at[pl.ds(base, BPW)], idx_v, sem),
            pltpu.async_copy(tab_h, tab_v, sem),
        ]
        for c in copies:
            c.wait()

        tmin_s = tab_v[pl.ds(0, L_)] * MIN_VAR_ + MIN_MEAN_
        tmax_s = tmin_s + (jnp.abs(tab_v[pl.ds(L_, L_)]) * DELTA_VAR_
                           + DELTA_MEAN_)

        @plsc.parallel_loop(0, BPW, step=L_, unroll=4)
        def _step(i):
            sl = pl.ds(i, L_)
            x0 = x0_v[sl]
            x1 = x1_v[sl]
            ee0 = e0_v[sl]
            ee1 = e1_v[sl]

            idxv = idx_v[sl]
            i0 = idxv + idxv
            i1 = i0 + 1
            tm0 = _dg(tmin_s, i0)
            tm1 = _dg(tmin_s, i1)
            tx0 = _dg(tmax_s, i0)
            tx1 = _dg(tmax_s, i1)

            t1m0 = jnp.abs(x0) * MIN_VAR_ + MIN_MEAN_
            t1m1 = jnp.abs(x1) * MIN_VAR_ + MIN_MEAN_
            t1d0 = jnp.abs(ee0) * MIN_VAR_ + MIN_MEAN_
            t1d1 = jnp.abs(ee1) * MIN_VAR_ + MIN_MEAN_
            t1x0 = t1m0 + t1d0
            t1x1 = t1m1 + t1d1

            md0 = jnp.minimum(t1x0, tx0) - jnp.maximum(t1m0, tm0)
            md1 = jnp.minimum(t1x1, tx1) - jnp.maximum(t1m1, tm1)
            disjoint = (md0 <= 0.0) | (md1 <= 0.0)
            meetprod = jnp.maximum(md0, EPS_) * jnp.maximum(md1, EPS_)
            d = t1d0 * t1d1
            j = (jnp.maximum(t1x0, tx0) - jnp.minimum(t1m0, tm0)) * \
                (jnp.maximum(t1x1, tx1) - jnp.minimum(t1m1, tm1))
            t2p = (tx0 - tm0) * (tx1 - tm1)

            ld = _vlog(d)
            pos_arg = jnp.where(disjoint,
                                jnp.maximum(j - d - t2p, EPS_ * j), meetprod)
            neg_arg = jnp.where(disjoint, d,
                                jnp.maximum(d - meetprod, EPS_ * d))
            pos_v[sl] = ld - _vlog(pos_arg)
            neg_v[sl] = ld - _vlog(neg_arg)

        outs = [
            pltpu.async_copy(pos_v, pos_h.at[pl.ds(base, BPW)], sem),
            pltpu.async_copy(neg_v, neg_h.at[pl.ds(base, BPW)], sem),
        ]
        for c in outs:
            c.wait()

    return pl.kernel(
        body,
        out_type=(jax.ShapeDtypeStruct((B,), jnp.float32),
                  jax.ShapeDtypeStruct((B,), jnp.float32)),
        mesh=mesh,
        scratch_types=[
            pltpu.VMEM((BPW,), jnp.float32),
            pltpu.VMEM((BPW,), jnp.float32),
            pltpu.VMEM((BPW,), jnp.float32),
            pltpu.VMEM((BPW,), jnp.float32),
            pltpu.VMEM((BPW,), jnp.int32),
            pltpu.VMEM((2 * L_,), jnp.float32),
            pltpu.VMEM((BPW,), jnp.float32),
            pltpu.VMEM((BPW,), jnp.float32),
            pltpu.SemaphoreType.DMA,
        ],
    )


def kernel(t1x, t2_embed1, t2_embed2, min_feature_embed, delta_feature_embed):
    B = t1x.shape[0]
    call = _make_sc_call(B)
    tab = jnp.concatenate([min_feature_embed.reshape(L_),
                           delta_feature_embed.reshape(L_)])
    return call(
        t1x.T.reshape(B * 2),
        t2_embed1.T.reshape(B * 2),
        t2_embed2.astype(jnp.int32),
        tab,
    )
